# baseline (speedup 1.0000x reference)
import functools

import jax
import jax.numpy as jnp
from jax import lax
from jax.experimental import pallas as pl
from jax.experimental.pallas import tpu as pltpu
from jax.experimental.pallas import tpu_sc as plsc

VOCAB = 100000
EDIM = 128
CTX = 200
HID = 128

_NC, _NS = 1, 16
_BPW = 16
_NFULL = CTX // _BPW
_TAILB = CTX - _BPW

_KT = 3200
_NK = (CTX * EDIM) // _KT
_VT = 4096
_NV = -(-VOCAB // _VT)
_NST = 8
_CNT = [_NV - (_NST - 1) * (_NV // _NST)] + [_NV // _NST] * (_NST - 1)
_START = [sum(_CNT[:s]) for s in range(_NST)]
_NP2 = _CNT[0]
_LASTW = VOCAB - (_NV - 1) * _VT


def _sc_gather_body(table_hbm, idx_hbm, w1_hbm, out_hbm, idx_v, rows_v, sem):
    del w1_hbm
    wid = lax.axis_index("s")

    @pl.when(wid <= _NFULL)
    def _():
        base = jnp.where(wid < _NFULL, wid * _BPW, _TAILB)
        pltpu.sync_copy(idx_hbm.at[pl.ds(base, _BPW)], idx_v)
        pltpu.async_copy(table_hbm.at[idx_v], rows_v, sem).wait()
        pltpu.sync_copy(rows_v, out_hbm.at[pl.ds(base, _BPW)])


@functools.lru_cache(maxsize=1)
def _sc_gather():
    return pl.kernel(
        _sc_gather_body,
        out_type=jax.ShapeDtypeStruct((CTX, EDIM), jnp.float32),
        mesh=plsc.VectorSubcoreMesh(core_axis_name="c", subcore_axis_name="s",
                                    num_cores=_NC),
        scratch_types=[
            pltpu.VMEM((_BPW,), jnp.int32),
            pltpu.VMEM((_BPW, EDIM), jnp.float32),
            pltpu.SemaphoreType.DMA,
        ],
    )


def _vocab_tile(tile, is_ragged, wref, b2_ref, out_ref,
                acc_ref, m_ref, s_ref, mh_ref):
    h = acc_ref[...]
    lt = lax.dot_general(h, wref[...],
                         (((1,), (1,)), ((), ())),
                         preferred_element_type=jnp.float32)
    lt = lt + b2_ref[...].reshape(1, _VT)
    if is_ragged:
        col = lax.broadcasted_iota(jnp.int32, (1, _VT), 1)
        lt = jnp.where(col < _LASTW, lt, -jnp.inf)

    m_old = m_ref[0, 0]
    m_new = jnp.maximum(m_old, jnp.max(lt))
    et = jnp.exp(lt - m_new)
    s_ref[0, 0] = s_ref[0, 0] * jnp.exp(m_old - m_new) + jnp.sum(et)
    m_ref[0, 0] = m_new
    mh_ref[tile, 0] = m_new
    if is_ragged:
        out_ref[0:1, pl.ds((_NV - 1) * _VT, _LASTW)] = et[:, :_LASTW]
    else:
        out_ref[0:1, pl.ds(tile * _VT, _VT)] = et


def _w1_copy(w1_hbm, w1buf, sems, c):
    return pltpu.make_async_copy(
        w1_hbm.at[pl.ds(c * _KT, _KT)], w1buf.at[c % 4], sems.at[c % 4])


def _mlp_body(*refs):
    (x_ref, w1_hbm, b1_ref), rest = refs[:3], refs[3:]
    w_refs, rest = rest[:_NST], rest[_NST:]
    b_refs, rest = rest[:_NST], rest[_NST:]
    out_ref, w1buf, sems, acc_ref, m_ref, s_ref, mh_ref = rest
    t = pl.program_id(0)

    @pl.when(t == 0)
    def _prime():
        for c in range(3):
            _w1_copy(w1_hbm, w1buf, sems, c).start()

    @pl.when(t < _NK)
    def _phase1():
        @pl.when(t == 0)
        def _():
            acc_ref[...] = jnp.zeros_like(acc_ref)
        _w1_copy(w1_hbm, w1buf, sems, t).wait()
        acc_ref[...] += jnp.dot(x_ref[0:1, pl.ds(t * _KT, _KT)],
                                w1buf[t % 4],
                                preferred_element_type=jnp.float32)

        @pl.when(t + 3 < _NK)
        def _():
            _w1_copy(w1_hbm, w1buf, sems, t + 3).start()

    @pl.when(t >= _NK)
    def _phase2():
        j = t - _NK

        @pl.when(j == 0)
        def _():
            acc_ref[...] = jnp.maximum(acc_ref[...] + b1_ref[...], 0.0)
            m_ref[0, 0] = -jnp.inf
            s_ref[0, 0] = 0.0

        base = (out_ref, acc_ref, m_ref, s_ref, mh_ref)
        _vocab_tile(_START[0] + j, False, w_refs[0], b_refs[0], *base)
        for s in range(1, _NST - 1):
            @pl.when(j < _CNT[s])
            def _(s=s):
                _vocab_tile(_START[s] + j, False, w_refs[s], b_refs[s], *base)

        sl = _NST - 1

        @pl.when(j < _CNT[sl] - 1)
        def _():
            _vocab_tile(_START[sl] + j, False, w_refs[sl], b_refs[sl], *base)

        @pl.when(j == _CNT[sl] - 1)
        def _():
            _vocab_tile(_START[sl] + j, True, w_refs[sl], b_refs[sl], *base)

        @pl.when(j == _NP2 - 1)
        def _finalize():
            m_fin = m_ref[0, 0]
            inv_s = 1.0 / s_ref[0, 0]
            for jj in range(_NV):
                c = jnp.exp(mh_ref[jj, 0] - m_fin) * inv_s
                w = _VT if jj < _NV - 1 else _LASTW
                sl2 = (slice(0, 1), pl.ds(jj * _VT, w))
                out_ref[sl2] = out_ref[sl2] * c


def _w2_spec(s):
    return pl.BlockSpec(
        (_VT, HID),
        lambda t, s=s: (_START[s] + jnp.clip(t - _NK, 0, _CNT[s] - 1), 0))


def _b2_spec(s):
    return pl.BlockSpec(
        (_VT,),
        lambda t, s=s: (_START[s] + jnp.clip(t - _NK, 0, _CNT[s] - 1),))


def _mlp_softmax(x, w1, b1, w2t, b2):
    return pl.pallas_call(
        _mlp_body,
        grid=(_NK + _NP2,),
        in_specs=[
            pl.BlockSpec((1, CTX * EDIM), lambda t: (0, 0)),
            pl.BlockSpec(memory_space=pltpu.MemorySpace.HBM),
            pl.BlockSpec((1, HID), lambda t: (0, 0)),
        ] + [_w2_spec(s) for s in range(_NST)]
          + [_b2_spec(s) for s in range(_NST)],
        out_specs=pl.BlockSpec((1, VOCAB), lambda t: (0, 0)),
        out_shape=jax.ShapeDtypeStruct((1, VOCAB), jnp.float32),
        scratch_shapes=[
            pltpu.VMEM((4, _KT, HID), jnp.float32),
            pltpu.SemaphoreType.DMA((4,)),
            pltpu.VMEM((1, HID), jnp.float32),
            pltpu.SMEM((1, 1), jnp.float32),
            pltpu.SMEM((1, 1), jnp.float32),
            pltpu.SMEM((_NV, 1), jnp.float32),
        ],
    )(x, w1, b1, *([w2t] * _NST), *([b2] * _NST))


def kernel(input, emb_table, W1, b1, W2, b2):
    embeds = _sc_gather()(emb_table, input.astype(jnp.int32), W1)
    x = embeds.reshape(1, CTX * EDIM)
    return _mlp_softmax(x, W1, b1.reshape(1, HID), W2.T, b2)


# speedup vs baseline: 1.0076x; 1.0076x over previous
import functools

import jax
import jax.numpy as jnp
from jax import lax
from jax.experimental import pallas as pl
from jax.experimental.pallas import tpu as pltpu
from jax.experimental.pallas import tpu_sc as plsc

VOCAB = 100000
EDIM = 128
CTX = 200
HID = 128

_NC, _NS = 1, 16
_BPW = 16
_NFULL = CTX // _BPW
_TAILB = CTX - _BPW

_KT = 3200
_NK = (CTX * EDIM) // _KT
_VT = 4096
_NV = -(-VOCAB // _VT)
_NST = 6
_CNT = [_NV - (_NST - 1) * (_NV // _NST)] + [_NV // _NST] * (_NST - 1)
_START = [sum(_CNT[:s]) for s in range(_NST)]
_NP2 = _CNT[0]
_LASTW = VOCAB - (_NV - 1) * _VT


def _sc_gather_body(table_hbm, idx_hbm, w1_hbm, out_hbm, idx_v, rows_v, sem):
    del w1_hbm
    wid = lax.axis_index("s")

    @pl.when(wid <= _NFULL)
    def _():
        base = jnp.where(wid < _NFULL, wid * _BPW, _TAILB)
        pltpu.sync_copy(idx_hbm.at[pl.ds(base, _BPW)], idx_v)
        pltpu.async_copy(table_hbm.at[idx_v], rows_v, sem).wait()
        pltpu.sync_copy(rows_v, out_hbm.at[pl.ds(base, _BPW)])


@functools.lru_cache(maxsize=1)
def _sc_gather():
    return pl.kernel(
        _sc_gather_body,
        out_type=jax.ShapeDtypeStruct((CTX, EDIM), jnp.float32),
        mesh=plsc.VectorSubcoreMesh(core_axis_name="c", subcore_axis_name="s",
                                    num_cores=_NC),
        scratch_types=[
            pltpu.VMEM((_BPW,), jnp.int32),
            pltpu.VMEM((_BPW, EDIM), jnp.float32),
            pltpu.SemaphoreType.DMA,
        ],
    )


def _vocab_tile(tile, is_ragged, wref, b2_ref, out_ref,
                acc_ref, m_ref, s_ref, mh_ref):
    h = acc_ref[...]
    lt = lax.dot_general(h, wref[...],
                         (((1,), (1,)), ((), ())),
                         preferred_element_type=jnp.float32)
    lt = lt + b2_ref[...].reshape(1, _VT)
    if is_ragged:
        col = lax.broadcasted_iota(jnp.int32, (1, _VT), 1)
        lt = jnp.where(col < _LASTW, lt, -jnp.inf)

    m_old = m_ref[0, 0]
    m_new = jnp.maximum(m_old, jnp.max(lt))
    et = jnp.exp(lt - m_new)
    s_ref[0, 0] = s_ref[0, 0] * jnp.exp(m_old - m_new) + jnp.sum(et)
    m_ref[0, 0] = m_new
    mh_ref[tile, 0] = m_new
    if is_ragged:
        out_ref[0:1, pl.ds((_NV - 1) * _VT, _LASTW)] = et[:, :_LASTW]
    else:
        out_ref[0:1, pl.ds(tile * _VT, _VT)] = et


def _w1_copy(w1_hbm, w1buf, sems, c):
    return pltpu.make_async_copy(
        w1_hbm.at[pl.ds(c * _KT, _KT)], w1buf.at[c % 4], sems.at[c % 4])


def _mlp_body(*refs):
    (x_ref, w1_hbm, b1_ref), rest = refs[:3], refs[3:]
    w_refs, rest = rest[:_NST], rest[_NST:]
    b_refs, rest = rest[:_NST], rest[_NST:]
    out_ref, w1buf, sems, acc_ref, m_ref, s_ref, mh_ref = rest
    t = pl.program_id(0)

    @pl.when(t == 0)
    def _prime():
        for c in range(3):
            _w1_copy(w1_hbm, w1buf, sems, c).start()

    @pl.when(t < _NK)
    def _phase1():
        @pl.when(t == 0)
        def _():
            acc_ref[...] = jnp.zeros_like(acc_ref)
        _w1_copy(w1_hbm, w1buf, sems, t).wait()
        acc_ref[...] += jnp.dot(x_ref[0:1, pl.ds(t * _KT, _KT)],
                                w1buf[t % 4],
                                preferred_element_type=jnp.float32)

        @pl.when(t + 3 < _NK)
        def _():
            _w1_copy(w1_hbm, w1buf, sems, t + 3).start()

    @pl.when(t >= _NK)
    def _phase2():
        j = t - _NK

        @pl.when(j == 0)
        def _():
            acc_ref[...] = jnp.maximum(acc_ref[...] + b1_ref[...], 0.0)
            m_ref[0, 0] = -jnp.inf
            s_ref[0, 0] = 0.0

        base = (out_ref, acc_ref, m_ref, s_ref, mh_ref)
        _vocab_tile(_START[0] + j, False, w_refs[0], b_refs[0], *base)
        for s in range(1, _NST - 1):
            @pl.when(j < _CNT[s])
            def _(s=s):
                _vocab_tile(_START[s] + j, False, w_refs[s], b_refs[s], *base)

        sl = _NST - 1

        @pl.when(j < _CNT[sl] - 1)
        def _():
            _vocab_tile(_START[sl] + j, False, w_refs[sl], b_refs[sl], *base)

        @pl.when(j == _CNT[sl] - 1)
        def _():
            _vocab_tile(_START[sl] + j, True, w_refs[sl], b_refs[sl], *base)

        @pl.when(j == _NP2 - 1)
        def _finalize():
            m_fin = m_ref[0, 0]
            inv_s = 1.0 / s_ref[0, 0]
            for jj in range(_NV):
                c = jnp.exp(mh_ref[jj, 0] - m_fin) * inv_s
                w = _VT if jj < _NV - 1 else _LASTW
                sl2 = (slice(0, 1), pl.ds(jj * _VT, w))
                out_ref[sl2] = out_ref[sl2] * c


def _w2_spec(s):
    return pl.BlockSpec(
        (_VT, HID),
        lambda t, s=s: (_START[s] + jnp.clip(t - _NK, 0, _CNT[s] - 1), 0))


def _b2_spec(s):
    return pl.BlockSpec(
        (_VT,),
        lambda t, s=s: (_START[s] + jnp.clip(t - _NK, 0, _CNT[s] - 1),))


def _mlp_softmax(x, w1, b1, w2t, b2):
    return pl.pallas_call(
        _mlp_body,
        grid=(_NK + _NP2,),
        in_specs=[
            pl.BlockSpec((1, CTX * EDIM), lambda t: (0, 0)),
            pl.BlockSpec(memory_space=pltpu.MemorySpace.HBM),
            pl.BlockSpec((1, HID), lambda t: (0, 0)),
        ] + [_w2_spec(s) for s in range(_NST)]
          + [_b2_spec(s) for s in range(_NST)],
        out_specs=pl.BlockSpec((1, VOCAB), lambda t: (0, 0)),
        out_shape=jax.ShapeDtypeStruct((1, VOCAB), jnp.float32),
        scratch_shapes=[
            pltpu.VMEM((4, _KT, HID), jnp.float32),
            pltpu.SemaphoreType.DMA((4,)),
            pltpu.VMEM((1, HID), jnp.float32),
            pltpu.SMEM((1, 1), jnp.float32),
            pltpu.SMEM((1, 1), jnp.float32),
            pltpu.SMEM((_NV, 1), jnp.float32),
        ],
    )(x, w1, b1, *([w2t] * _NST), *([b2] * _NST))


def kernel(input, emb_table, W1, b1, W2, b2):
    embeds = _sc_gather()(emb_table, input.astype(jnp.int32), W1)
    x = embeds.reshape(1, CTX * EDIM)
    return _mlp_softmax(x, W1, b1.reshape(1, HID), W2.T, b2)


# speedup vs baseline: 1.0194x; 1.0117x over previous
import functools

import jax
import jax.numpy as jnp
from jax import lax
from jax.experimental import pallas as pl
from jax.experimental.pallas import tpu as pltpu
from jax.experimental.pallas import tpu_sc as plsc

VOCAB = 100000
EDIM = 128
CTX = 200
HID = 128

_NC, _NS = 1, 16
_BPW = 16
_NFULL = CTX // _BPW
_TAILB = CTX - _BPW

_KT = 6400
_NK = (CTX * EDIM) // _KT
_VT = 4096
_NV = -(-VOCAB // _VT)
_NST = 6
_CNT = [_NV - (_NST - 1) * (_NV // _NST)] + [_NV // _NST] * (_NST - 1)
_START = [sum(_CNT[:s]) for s in range(_NST)]
_NP2 = _CNT[0]
_LASTW = VOCAB - (_NV - 1) * _VT


def _sc_gather_body(table_hbm, idx_hbm, w1_hbm, out_hbm, idx_v, rows_v, sem):
    del w1_hbm
    wid = lax.axis_index("s")

    @pl.when(wid <= _NFULL)
    def _():
        base = jnp.where(wid < _NFULL, wid * _BPW, _TAILB)
        pltpu.sync_copy(idx_hbm.at[pl.ds(base, _BPW)], idx_v)
        pltpu.async_copy(table_hbm.at[idx_v], rows_v, sem).wait()
        pltpu.sync_copy(rows_v, out_hbm.at[pl.ds(base, _BPW)])


@functools.lru_cache(maxsize=1)
def _sc_gather():
    return pl.kernel(
        _sc_gather_body,
        out_type=jax.ShapeDtypeStruct((CTX, EDIM), jnp.float32),
        mesh=plsc.VectorSubcoreMesh(core_axis_name="c", subcore_axis_name="s",
                                    num_cores=_NC),
        scratch_types=[
            pltpu.VMEM((_BPW,), jnp.int32),
            pltpu.VMEM((_BPW, EDIM), jnp.float32),
            pltpu.SemaphoreType.DMA,
        ],
    )


def _vocab_tile(tile, is_ragged, wref, b2_ref, out_ref,
                acc_ref, m_ref, s_ref, mh_ref):
    h = acc_ref[...]
    lt = lax.dot_general(h, wref[...],
                         (((1,), (1,)), ((), ())),
                         preferred_element_type=jnp.float32)
    lt = lt + b2_ref[...].reshape(1, _VT)
    if is_ragged:
        col = lax.broadcasted_iota(jnp.int32, (1, _VT), 1)
        lt = jnp.where(col < _LASTW, lt, -jnp.inf)

    m_old = m_ref[0, 0]
    m_new = jnp.maximum(m_old, jnp.max(lt))
    et = jnp.exp(lt - m_new)
    s_ref[0, 0] = s_ref[0, 0] * jnp.exp(m_old - m_new) + jnp.sum(et)
    m_ref[0, 0] = m_new
    mh_ref[tile, 0] = m_new
    if is_ragged:
        out_ref[0:1, pl.ds((_NV - 1) * _VT, _LASTW)] = et[:, :_LASTW]
    else:
        out_ref[0:1, pl.ds(tile * _VT, _VT)] = et


def _w1_copy(w1_hbm, w1buf, sems, c):
    return pltpu.make_async_copy(
        w1_hbm.at[pl.ds(c * _KT, _KT)], w1buf.at[c % 4], sems.at[c % 4])


def _mlp_body(*refs):
    (x_ref, w1_hbm, b1_ref), rest = refs[:3], refs[3:]
    w_refs, rest = rest[:_NST], rest[_NST:]
    b_refs, rest = rest[:_NST], rest[_NST:]
    out_ref, w1buf, sems, acc_ref, m_ref, s_ref, mh_ref = rest
    t = pl.program_id(0)

    @pl.when(t == 0)
    def _prime():
        for c in range(3):
            _w1_copy(w1_hbm, w1buf, sems, c).start()

    @pl.when(t < _NK)
    def _phase1():
        @pl.when(t == 0)
        def _():
            acc_ref[...] = jnp.zeros_like(acc_ref)
        _w1_copy(w1_hbm, w1buf, sems, t).wait()
        acc_ref[...] += jnp.dot(x_ref[0:1, pl.ds(t * _KT, _KT)],
                                w1buf[t % 4],
                                preferred_element_type=jnp.float32)

        @pl.when(t + 3 < _NK)
        def _():
            _w1_copy(w1_hbm, w1buf, sems, t + 3).start()

    @pl.when(t >= _NK)
    def _phase2():
        j = t - _NK

        @pl.when(j == 0)
        def _():
            acc_ref[...] = jnp.maximum(acc_ref[...] + b1_ref[...], 0.0)
            m_ref[0, 0] = -jnp.inf
            s_ref[0, 0] = 0.0

        base = (out_ref, acc_ref, m_ref, s_ref, mh_ref)
        _vocab_tile(_START[0] + j, False, w_refs[0], b_refs[0], *base)
        for s in range(1, _NST - 1):
            @pl.when(j < _CNT[s])
            def _(s=s):
                _vocab_tile(_START[s] + j, False, w_refs[s], b_refs[s], *base)

        sl = _NST - 1

        @pl.when(j < _CNT[sl] - 1)
        def _():
            _vocab_tile(_START[sl] + j, False, w_refs[sl], b_refs[sl], *base)

        @pl.when(j == _CNT[sl] - 1)
        def _():
            _vocab_tile(_START[sl] + j, True, w_refs[sl], b_refs[sl], *base)

        @pl.when(j == _NP2 - 1)
        def _finalize():
            m_fin = m_ref[0, 0]
            inv_s = 1.0 / s_ref[0, 0]
            for jj in range(_NV):
                c = jnp.exp(mh_ref[jj, 0] - m_fin) * inv_s
                w = _VT if jj < _NV - 1 else _LASTW
                sl2 = (slice(0, 1), pl.ds(jj * _VT, w))
                out_ref[sl2] = out_ref[sl2] * c


def _w2_spec(s):
    return pl.BlockSpec(
        (_VT, HID),
        lambda t, s=s: (_START[s] + jnp.clip(t - _NK, 0, _CNT[s] - 1), 0))


def _b2_spec(s):
    return pl.BlockSpec(
        (_VT,),
        lambda t, s=s: (_START[s] + jnp.clip(t - _NK, 0, _CNT[s] - 1),))


def _mlp_softmax(x, w1, b1, w2t, b2):
    return pl.pallas_call(
        _mlp_body,
        grid=(_NK + _NP2,),
        in_specs=[
            pl.BlockSpec((1, CTX * EDIM), lambda t: (0, 0)),
            pl.BlockSpec(memory_space=pltpu.MemorySpace.HBM),
            pl.BlockSpec((1, HID), lambda t: (0, 0)),
        ] + [_w2_spec(s) for s in range(_NST)]
          + [_b2_spec(s) for s in range(_NST)],
        out_specs=pl.BlockSpec((1, VOCAB), lambda t: (0, 0)),
        out_shape=jax.ShapeDtypeStruct((1, VOCAB), jnp.float32),
        scratch_shapes=[
            pltpu.VMEM((4, _KT, HID), jnp.float32),
            pltpu.SemaphoreType.DMA((4,)),
            pltpu.VMEM((1, HID), jnp.float32),
            pltpu.SMEM((1, 1), jnp.float32),
            pltpu.SMEM((1, 1), jnp.float32),
            pltpu.SMEM((_NV, 1), jnp.float32),
        ],
    )(x, w1, b1, *([w2t] * _NST), *([b2] * _NST))


def kernel(input, emb_table, W1, b1, W2, b2):
    embeds = _sc_gather()(emb_table, input.astype(jnp.int32), W1)
    x = embeds.reshape(1, CTX * EDIM)
    return _mlp_softmax(x, W1, b1.reshape(1, HID), W2.T, b2)
